# Initial kernel scaffold; baseline (speedup 1.0000x reference)
import jax
import jax.numpy as jnp
from jax.experimental import pallas as pl


def kernel(xyz, features, W1, b1, g1, be1, W2, b2, g2, be2, W3, b3, g3, be3):
    raise NotImplementedError("write your pallas kernel here")


# speedup vs baseline: 11.8867x; 11.8867x over previous
import functools

import jax
import jax.numpy as jnp
from jax import lax
from jax.experimental import pallas as pl
from jax.experimental.pallas import tpu as pltpu
from jax.experimental.pallas import tpu_sc as plsc

B, N, M, K, C = 8, 4096, 512, 32, 128
R2 = 0.2 ** 2
P = B * M * K
BN_N = float(P)
EPS = 1e-5
C3 = 256


def _fps_body(xt_ref, nxt_ref):
    x = xt_ref[0]
    y = xt_ref[1]
    z = xt_ref[2]
    lanes_n = lax.broadcasted_iota(jnp.int32, (B, N), 1)
    lanes_m = lax.broadcasted_iota(jnp.int32, (B, M), 1)

    def body(i, carry):
        dist, far, ax, ay, az = carry
        sel = lanes_n == far
        cx = jnp.sum(jnp.where(sel, x, 0.0), axis=1, keepdims=True)
        cy = jnp.sum(jnp.where(sel, y, 0.0), axis=1, keepdims=True)
        cz = jnp.sum(jnp.where(sel, z, 0.0), axis=1, keepdims=True)
        put = lanes_m == i
        ax = jnp.where(put, cx, ax)
        ay = jnp.where(put, cy, ay)
        az = jnp.where(put, cz, az)
        d = (x - cx) ** 2 + (y - cy) ** 2 + (z - cz) ** 2
        dist = jnp.minimum(dist, d)
        mx = jnp.max(dist, axis=1, keepdims=True)
        far = jnp.min(jnp.where(dist == mx, lanes_n, N), axis=1, keepdims=True)
        return dist, far, ax, ay, az

    dist0 = jnp.full((B, N), 1e10, jnp.float32)
    far0 = jnp.zeros((B, 1), jnp.int32)
    a0 = jnp.zeros((B, M), jnp.float32)
    _, _, ax, ay, az = lax.fori_loop(0, M, body, (dist0, far0, a0, a0, a0))
    nxt_ref[0] = ax
    nxt_ref[1] = ay
    nxt_ref[2] = az


def _run_fps(xt):
    return pl.pallas_call(
        _fps_body,
        out_shape=jax.ShapeDtypeStruct((3, B, M), jnp.float32),
    )(xt)


def _ball_body(nxyz_ref, xt_ref, w1t_ref, idx_ref, v_ref, dm_ref):
    b = pl.program_id(0)
    nx = nxyz_ref[0]
    v = jnp.zeros((M, C), jnp.float32)
    for j in range(3):
        v = v + nx[:, j:j + 1] * w1t_ref[j:j + 1, :]
    v_ref[0] = v
    def bf(t):
        return t.astype(jnp.bfloat16).astype(jnp.float32)

    n0, n1, n2 = nx[:, 0:1], nx[:, 1:2], nx[:, 2:3]
    p0, p1, p2 = xt_ref[0, 0], xt_ref[1, 0], xt_ref[2, 0]
    cross = (bf(n0) * bf(p0) + bf(n1) * bf(p1)) + bf(n2) * bf(p2)
    d = -2.0 * cross
    d = d + ((n0 * n0 + n1 * n1) + n2 * n2)
    d = d + ((p0 * p0 + p1 * p1) + p2 * p2)
    d = jnp.maximum(d, 0.0)
    dm_ref[...] = jnp.where(d > R2, jnp.inf, d)
    lanes_n = lax.broadcasted_iota(jnp.int32, (M, N), 1)
    lanes_k = lax.broadcasted_iota(jnp.int32, (M, K), 1)
    base = b * N

    def body(k, carry):
        acc, idx0 = carry
        dmv = dm_ref[...]
        mn = jnp.min(dmv, axis=1, keepdims=True)
        sel = jnp.min(jnp.where(dmv == mn, lanes_n, N), axis=1, keepdims=True)
        idx0 = jnp.where(k == 0, sel, idx0)
        chosen = jnp.where(mn == jnp.inf, idx0, sel) + base
        acc = jnp.where(lanes_k == k, chosen, acc)
        dm_ref[...] = jnp.where(lanes_n == sel, jnp.inf, dmv)
        return acc, idx0

    acc0 = jnp.zeros((M, K), jnp.int32)
    acc, _ = lax.fori_loop(0, K, body, (acc0, jnp.zeros((M, 1), jnp.int32)))
    idx_ref[0] = acc


def _run_ball(new_xyz, xt, w1t):
    xt4 = xt.reshape(3, B, 1, N)
    return pl.pallas_call(
        _ball_body,
        grid=(B,),
        in_specs=[
            pl.BlockSpec((1, M, 3), lambda i: (i, 0, 0)),
            pl.BlockSpec((3, 1, 1, N), lambda i: (0, i, 0, 0)),
            pl.BlockSpec((131, C), lambda i: (0, 0)),
        ],
        out_specs=[
            pl.BlockSpec((1, M, K), lambda i: (i, 0, 0)),
            pl.BlockSpec((1, M, C), lambda i: (i, 0, 0)),
        ],
        out_shape=[
            jax.ShapeDtypeStruct((B, M, K), jnp.int32),
            jax.ShapeDtypeStruct((B, M, C), jnp.float32),
        ],
        scratch_shapes=[pltpu.VMEM((M, N), jnp.float32)],
    )(new_xyz, xt4, w1t)


def _mm1_body(x_ref, w_ref, b_ref, o_ref):
    o_ref[...] = (
        jnp.dot(x_ref[...], w_ref[...], preferred_element_type=jnp.float32)
        + b_ref[...]
    )


def _run_mm1(xall, w1t, b1r):
    nblk = 8
    rows = (B * N) // nblk
    return pl.pallas_call(
        _mm1_body,
        grid=(nblk,),
        in_specs=[
            pl.BlockSpec((rows, 131), lambda i: (i, 0)),
            pl.BlockSpec((131, C), lambda i: (0, 0)),
            pl.BlockSpec((1, C), lambda i: (0, 0)),
        ],
        out_specs=pl.BlockSpec((rows, C), lambda i: (i, 0)),
        out_shape=jax.ShapeDtypeStruct((B * N, C), jnp.float32),
    )(xall, w1t, b1r)


_SC_NW = 32
_SC_CHUNK = 128
_SC_NCHUNK = P // _SC_NW // _SC_CHUNK


def _gather_rows(g2d, idx2d):
    mesh = plsc.VectorSubcoreMesh(core_axis_name="c", subcore_axis_name="s")

    @functools.partial(
        pl.kernel,
        mesh=mesh,
        out_type=jax.ShapeDtypeStruct((P, C), jnp.float32),
        scratch_types=[
            pltpu.VMEM((_SC_NCHUNK, _SC_CHUNK), jnp.int32),
            pltpu.VMEM((_SC_CHUNK, C), jnp.float32),
            pltpu.SemaphoreType.DMA,
        ],
    )
    def k(g_hbm, idx_hbm, out_hbm, idx_v, rows_v, sem):
        wid = lax.axis_index("s") * 2 + lax.axis_index("c")
        chunk0 = wid * _SC_NCHUNK
        pltpu.sync_copy(idx_hbm.at[pl.ds(chunk0, _SC_NCHUNK)], idx_v)

        def body(j, _):
            pltpu.async_copy(g_hbm.at[idx_v.at[j]], rows_v, sem).wait()
            pltpu.sync_copy(
                rows_v, out_hbm.at[pl.ds((chunk0 + j) * _SC_CHUNK, _SC_CHUNK)]
            )
            return 0

        lax.fori_loop(0, _SC_NCHUNK, body, 0)

    return k(g2d, idx2d)


def _stats1_body(y_ref, v_ref, s_ref):
    y = y_ref[...] - v_ref[...][:, None, :]

    @pl.when(pl.program_id(0) == 0)
    def _():
        s_ref[...] = jnp.zeros_like(s_ref)

    s_ref[0:1, :] += jnp.sum(y, axis=(0, 1))[None, :]
    s_ref[1:2, :] += jnp.sum(y * y, axis=(0, 1))[None, :]


def _run_stats1(y1_3d, v2d):
    nblk = 32
    g = (B * M) // nblk
    return pl.pallas_call(
        _stats1_body,
        grid=(nblk,),
        in_specs=[
            pl.BlockSpec((g, K, C), lambda i: (i, 0, 0)),
            pl.BlockSpec((g, C), lambda i: (i, 0)),
        ],
        out_specs=pl.BlockSpec((8, C), lambda i: (0, 0)),
        out_shape=jax.ShapeDtypeStruct((8, C), jnp.float32),
    )(y1_3d, v2d)


def _bn_coeffs(s_ref, g_ref, be_ref, cols):
    mu = s_ref[0:1, :] / BN_N
    var = s_ref[1:2, :] / BN_N - mu * mu
    inv = g_ref[...] / jnp.sqrt(var + EPS)
    cc = be_ref[...] - mu * inv
    return inv.reshape(1, 1, cols), cc.reshape(1, 1, cols)


def _mm2_body(y_ref, v_ref, s_ref, g_ref, be_ref, w_ref, b2_ref, o_ref, s2_ref):
    inv, cc = _bn_coeffs(s_ref, g_ref, be_ref, C)
    y = y_ref[...] - v_ref[...][:, None, :]
    x = jnp.maximum(y * inv + cc, 0.0)
    xf = x.reshape(y.shape[0] * K, C)
    y2 = (
        jnp.dot(xf, w_ref[...], preferred_element_type=jnp.float32)
        + b2_ref[...]
    )
    o_ref[...] = y2

    @pl.when(pl.program_id(0) == 0)
    def _():
        s2_ref[...] = jnp.zeros_like(s2_ref)

    s2_ref[0:1, :] += jnp.sum(y2, axis=0)[None, :]
    s2_ref[1:2, :] += jnp.sum(y2 * y2, axis=0)[None, :]


def _run_mm2(y1_3d, v2d, s1, g1r, be1r, w2t, b2r):
    nblk = 32
    g = (B * M) // nblk
    return pl.pallas_call(
        _mm2_body,
        grid=(nblk,),
        in_specs=[
            pl.BlockSpec((g, K, C), lambda i: (i, 0, 0)),
            pl.BlockSpec((g, C), lambda i: (i, 0)),
            pl.BlockSpec((8, C), lambda i: (0, 0)),
            pl.BlockSpec((1, C), lambda i: (0, 0)),
            pl.BlockSpec((1, C), lambda i: (0, 0)),
            pl.BlockSpec((C, C), lambda i: (0, 0)),
            pl.BlockSpec((1, C), lambda i: (0, 0)),
        ],
        out_specs=[
            pl.BlockSpec((g * K, C), lambda i: (i, 0)),
            pl.BlockSpec((8, C), lambda i: (0, 0)),
        ],
        out_shape=[
            jax.ShapeDtypeStruct((P, C), jnp.float32),
            jax.ShapeDtypeStruct((8, C), jnp.float32),
        ],
    )(y1_3d, v2d, s1, g1r, be1r, w2t, b2r)


def _mm3_body(y_ref, s_ref, g_ref, be_ref, w_ref, b3_ref, o_ref, s3_ref):
    mu = s_ref[0:1, :] / BN_N
    var = s_ref[1:2, :] / BN_N - mu * mu
    inv = g_ref[...] / jnp.sqrt(var + EPS)
    cc = be_ref[...] - mu * inv
    x = jnp.maximum(y_ref[...] * inv + cc, 0.0)
    y3 = (
        jnp.dot(x, w_ref[...], preferred_element_type=jnp.float32)
        + b3_ref[...]
    )
    o_ref[...] = y3

    @pl.when(pl.program_id(0) == 0)
    def _():
        s3_ref[...] = jnp.zeros_like(s3_ref)

    s3_ref[0:1, :] += jnp.sum(y3, axis=0)[None, :]
    s3_ref[1:2, :] += jnp.sum(y3 * y3, axis=0)[None, :]


def _run_mm3(y2, s2, g2r, be2r, w3t, b3r):
    nblk = 32
    rows = P // nblk
    return pl.pallas_call(
        _mm3_body,
        grid=(nblk,),
        in_specs=[
            pl.BlockSpec((rows, C), lambda i: (i, 0)),
            pl.BlockSpec((8, C), lambda i: (0, 0)),
            pl.BlockSpec((1, C), lambda i: (0, 0)),
            pl.BlockSpec((1, C), lambda i: (0, 0)),
            pl.BlockSpec((C, C3), lambda i: (0, 0)),
            pl.BlockSpec((1, C3), lambda i: (0, 0)),
        ],
        out_specs=[
            pl.BlockSpec((rows, C3), lambda i: (i, 0)),
            pl.BlockSpec((8, C3), lambda i: (0, 0)),
        ],
        out_shape=[
            jax.ShapeDtypeStruct((P, C3), jnp.float32),
            jax.ShapeDtypeStruct((8, C3), jnp.float32),
        ],
    )(y2, s2, g2r, be2r, w3t, b3r)


def _final_body(y_ref, s_ref, g_ref, be_ref, o_ref):
    mu = s_ref[0:1, :] / BN_N
    var = s_ref[1:2, :] / BN_N - mu * mu
    inv = g_ref[...] / jnp.sqrt(var + EPS)
    cc = be_ref[...] - mu * inv
    z = jnp.maximum(y_ref[...] * inv.reshape(1, 1, C3) + cc.reshape(1, 1, C3), 0.0)
    o_ref[...] = jnp.max(z, axis=1)


def _run_final(y3_3d, s3, g3r, be3r):
    nblk = 32
    g = (B * M) // nblk
    return pl.pallas_call(
        _final_body,
        grid=(nblk,),
        in_specs=[
            pl.BlockSpec((g, K, C3), lambda i: (i, 0, 0)),
            pl.BlockSpec((8, C3), lambda i: (0, 0)),
            pl.BlockSpec((1, C3), lambda i: (0, 0)),
            pl.BlockSpec((1, C3), lambda i: (0, 0)),
        ],
        out_specs=pl.BlockSpec((g, C3), lambda i: (i, 0)),
        out_shape=jax.ShapeDtypeStruct((B * M, C3), jnp.float32),
    )(y3_3d, s3, g3r, be3r)


def kernel(xyz, features, W1, b1, g1, be1, W2, b2, g2, be2, W3, b3, g3, be3):
    xt = jnp.transpose(xyz, (2, 0, 1))
    nxt = _run_fps(xt)
    new_xyz = jnp.transpose(nxt, (1, 2, 0))

    w1t = W1.T
    xall = jnp.concatenate([xyz, features], axis=-1).reshape(B * N, 131)
    g2d = _run_mm1(xall, w1t, b1.reshape(1, C))

    idx, v = _run_ball(new_xyz, xt, w1t)

    y1 = _gather_rows(g2d, idx.reshape(P // 128, 128))
    y1_3d = y1.reshape(B * M, K, C)
    v2d = v.reshape(B * M, C)

    s1 = _run_stats1(y1_3d, v2d)
    y2, s2 = _run_mm2(
        y1_3d, v2d, s1, g1.reshape(1, C), be1.reshape(1, C), W2.T,
        b2.reshape(1, C),
    )
    y3, s3 = _run_mm3(
        y2, s2, g2.reshape(1, C), be2.reshape(1, C), W3.T, b3.reshape(1, C3)
    )
    nf = _run_final(
        y3.reshape(B * M, K, C3), s3, g3.reshape(1, C3), be3.reshape(1, C3)
    )
    return new_xyz, nf.reshape(B, M, C3)


# speedup vs baseline: 12.3607x; 1.0399x over previous
import functools

import jax
import jax.numpy as jnp
from jax import lax
from jax.experimental import pallas as pl
from jax.experimental.pallas import tpu as pltpu
from jax.experimental.pallas import tpu_sc as plsc

B, N, M, K, C = 8, 4096, 512, 32, 128
R2 = 0.2 ** 2
P = B * M * K
BN_N = float(P)
EPS = 1e-5
C3 = 256


def _fps_body(xt_ref, nxt_ref):
    x = xt_ref[0]
    y = xt_ref[1]
    z = xt_ref[2]
    lanes_n = lax.broadcasted_iota(jnp.int32, (B, N), 1)
    lanes_m = lax.broadcasted_iota(jnp.int32, (B, M), 1)

    def body(i, carry):
        dist, far, ax, ay, az = carry
        sel = lanes_n == far
        cx = jnp.sum(jnp.where(sel, x, 0.0), axis=1, keepdims=True)
        cy = jnp.sum(jnp.where(sel, y, 0.0), axis=1, keepdims=True)
        cz = jnp.sum(jnp.where(sel, z, 0.0), axis=1, keepdims=True)
        put = lanes_m == i
        ax = jnp.where(put, cx, ax)
        ay = jnp.where(put, cy, ay)
        az = jnp.where(put, cz, az)
        d = (x - cx) ** 2 + (y - cy) ** 2 + (z - cz) ** 2
        dist = jnp.minimum(dist, d)
        mx = jnp.max(dist, axis=1, keepdims=True)
        far = jnp.min(jnp.where(dist == mx, lanes_n, N), axis=1, keepdims=True)
        return dist, far, ax, ay, az

    dist0 = jnp.full((B, N), 1e10, jnp.float32)
    far0 = jnp.zeros((B, 1), jnp.int32)
    a0 = jnp.zeros((B, M), jnp.float32)
    _, _, ax, ay, az = lax.fori_loop(0, M, body, (dist0, far0, a0, a0, a0))
    nxt_ref[0] = ax
    nxt_ref[1] = ay
    nxt_ref[2] = az


def _run_fps(xt):
    return pl.pallas_call(
        _fps_body,
        out_shape=jax.ShapeDtypeStruct((3, B, M), jnp.float32),
    )(xt)


def _ball_body(nxyz_ref, xt_ref, w1t_ref, idx_ref, v_ref, dm_ref):
    b = pl.program_id(0)
    nx = nxyz_ref[0]
    v = jnp.zeros((M, C), jnp.float32)
    for j in range(3):
        v = v + nx[:, j:j + 1] * w1t_ref[j:j + 1, :]
    v_ref[0] = v
    def bf(t):
        return t.astype(jnp.bfloat16).astype(jnp.float32)

    n0, n1, n2 = nx[:, 0:1], nx[:, 1:2], nx[:, 2:3]
    p0, p1, p2 = xt_ref[0, 0], xt_ref[1, 0], xt_ref[2, 0]
    cross = (bf(n0) * bf(p0) + bf(n1) * bf(p1)) + bf(n2) * bf(p2)
    d = -2.0 * cross
    d = d + ((n0 * n0 + n1 * n1) + n2 * n2)
    d = d + ((p0 * p0 + p1 * p1) + p2 * p2)
    d = jnp.maximum(d, 0.0)
    dm_ref[...] = jnp.where(d > R2, jnp.inf, d)
    lanes_n = lax.broadcasted_iota(jnp.int32, (M, N), 1)
    lanes_k = lax.broadcasted_iota(jnp.int32, (M, K), 1)
    base = b * N

    NSUB = 4

    def body(k, carry):
        acc, idx0 = carry
        dmv = dm_ref[...]
        for t in range(NSUB):
            mn = jnp.min(dmv, axis=1, keepdims=True)
            sel = jnp.min(
                jnp.where(dmv == mn, lanes_n, N), axis=1, keepdims=True)
            idx0 = jnp.where((k == 0) & (t == 0), sel, idx0)
            chosen = jnp.where(mn == jnp.inf, idx0, sel) + base
            acc = jnp.where(lanes_k == k * NSUB + t, chosen, acc)
            dmv = jnp.where(lanes_n == sel, jnp.inf, dmv)
        dm_ref[...] = dmv
        return acc, idx0

    acc0 = jnp.zeros((M, K), jnp.int32)
    acc, _ = lax.fori_loop(
        0, K // NSUB, body, (acc0, jnp.zeros((M, 1), jnp.int32)))
    idx_ref[0] = acc


def _run_ball(new_xyz, xt, w1t):
    xt4 = xt.reshape(3, B, 1, N)
    return pl.pallas_call(
        _ball_body,
        grid=(B,),
        in_specs=[
            pl.BlockSpec((1, M, 3), lambda i: (i, 0, 0)),
            pl.BlockSpec((3, 1, 1, N), lambda i: (0, i, 0, 0)),
            pl.BlockSpec((131, C), lambda i: (0, 0)),
        ],
        out_specs=[
            pl.BlockSpec((1, M, K), lambda i: (i, 0, 0)),
            pl.BlockSpec((1, M, C), lambda i: (i, 0, 0)),
        ],
        out_shape=[
            jax.ShapeDtypeStruct((B, M, K), jnp.int32),
            jax.ShapeDtypeStruct((B, M, C), jnp.float32),
        ],
        scratch_shapes=[pltpu.VMEM((M, N), jnp.float32)],
    )(new_xyz, xt4, w1t)


def _mm1_body(x_ref, w_ref, b_ref, o_ref):
    o_ref[...] = (
        jnp.dot(x_ref[...], w_ref[...], preferred_element_type=jnp.float32)
        + b_ref[...]
    )


def _run_mm1(xall, w1t, b1r):
    nblk = 8
    rows = (B * N) // nblk
    return pl.pallas_call(
        _mm1_body,
        grid=(nblk,),
        in_specs=[
            pl.BlockSpec((rows, 131), lambda i: (i, 0)),
            pl.BlockSpec((131, C), lambda i: (0, 0)),
            pl.BlockSpec((1, C), lambda i: (0, 0)),
        ],
        out_specs=pl.BlockSpec((rows, C), lambda i: (i, 0)),
        out_shape=jax.ShapeDtypeStruct((B * N, C), jnp.float32),
    )(xall, w1t, b1r)


_SC_NW = 32
_SC_CHUNK = 128
_SC_NCHUNK = P // _SC_NW // _SC_CHUNK


def _gather_rows(g2d, idx2d):
    mesh = plsc.VectorSubcoreMesh(core_axis_name="c", subcore_axis_name="s")

    @functools.partial(
        pl.kernel,
        mesh=mesh,
        out_type=jax.ShapeDtypeStruct((P, C), jnp.float32),
        scratch_types=[
            pltpu.VMEM((_SC_NCHUNK, _SC_CHUNK), jnp.int32),
            pltpu.VMEM((_SC_CHUNK, C), jnp.float32),
            pltpu.SemaphoreType.DMA,
        ],
    )
    def k(g_hbm, idx_hbm, out_hbm, idx_v, rows_v, sem):
        wid = lax.axis_index("s") * 2 + lax.axis_index("c")
        chunk0 = wid * _SC_NCHUNK
        pltpu.sync_copy(idx_hbm.at[pl.ds(chunk0, _SC_NCHUNK)], idx_v)

        def body(j, _):
            pltpu.async_copy(g_hbm.at[idx_v.at[j]], rows_v, sem).wait()
            pltpu.sync_copy(
                rows_v, out_hbm.at[pl.ds((chunk0 + j) * _SC_CHUNK, _SC_CHUNK)]
            )
            return 0

        lax.fori_loop(0, _SC_NCHUNK, body, 0)

    return k(g2d, idx2d)


def _stats1_body(y_ref, v_ref, s_ref):
    y = y_ref[...] - v_ref[...][:, None, :]

    @pl.when(pl.program_id(0) == 0)
    def _():
        s_ref[...] = jnp.zeros_like(s_ref)

    s_ref[0:1, :] += jnp.sum(y, axis=(0, 1))[None, :]
    s_ref[1:2, :] += jnp.sum(y * y, axis=(0, 1))[None, :]


def _run_stats1(y1_3d, v2d):
    nblk = 32
    g = (B * M) // nblk
    return pl.pallas_call(
        _stats1_body,
        grid=(nblk,),
        in_specs=[
            pl.BlockSpec((g, K, C), lambda i: (i, 0, 0)),
            pl.BlockSpec((g, C), lambda i: (i, 0)),
        ],
        out_specs=pl.BlockSpec((8, C), lambda i: (0, 0)),
        out_shape=jax.ShapeDtypeStruct((8, C), jnp.float32),
    )(y1_3d, v2d)


def _bn_coeffs(s_ref, g_ref, be_ref, cols):
    mu = s_ref[0:1, :] / BN_N
    var = s_ref[1:2, :] / BN_N - mu * mu
    inv = g_ref[...] / jnp.sqrt(var + EPS)
    cc = be_ref[...] - mu * inv
    return inv.reshape(1, 1, cols), cc.reshape(1, 1, cols)


def _mm2_body(y_ref, v_ref, s_ref, g_ref, be_ref, w_ref, b2_ref, o_ref, s2_ref):
    inv, cc = _bn_coeffs(s_ref, g_ref, be_ref, C)
    y = y_ref[...] - v_ref[...][:, None, :]
    x = jnp.maximum(y * inv + cc, 0.0)
    xf = x.reshape(y.shape[0] * K, C)
    y2 = (
        jnp.dot(xf, w_ref[...], preferred_element_type=jnp.float32)
        + b2_ref[...]
    )
    o_ref[...] = y2

    @pl.when(pl.program_id(0) == 0)
    def _():
        s2_ref[...] = jnp.zeros_like(s2_ref)

    s2_ref[0:1, :] += jnp.sum(y2, axis=0)[None, :]
    s2_ref[1:2, :] += jnp.sum(y2 * y2, axis=0)[None, :]


def _run_mm2(y1_3d, v2d, s1, g1r, be1r, w2t, b2r):
    nblk = 32
    g = (B * M) // nblk
    return pl.pallas_call(
        _mm2_body,
        grid=(nblk,),
        in_specs=[
            pl.BlockSpec((g, K, C), lambda i: (i, 0, 0)),
            pl.BlockSpec((g, C), lambda i: (i, 0)),
            pl.BlockSpec((8, C), lambda i: (0, 0)),
            pl.BlockSpec((1, C), lambda i: (0, 0)),
            pl.BlockSpec((1, C), lambda i: (0, 0)),
            pl.BlockSpec((C, C), lambda i: (0, 0)),
            pl.BlockSpec((1, C), lambda i: (0, 0)),
        ],
        out_specs=[
            pl.BlockSpec((g * K, C), lambda i: (i, 0)),
            pl.BlockSpec((8, C), lambda i: (0, 0)),
        ],
        out_shape=[
            jax.ShapeDtypeStruct((P, C), jnp.float32),
            jax.ShapeDtypeStruct((8, C), jnp.float32),
        ],
    )(y1_3d, v2d, s1, g1r, be1r, w2t, b2r)


def _mm3_body(y_ref, s_ref, g_ref, be_ref, w_ref, b3_ref, o_ref, s3_ref):
    mu = s_ref[0:1, :] / BN_N
    var = s_ref[1:2, :] / BN_N - mu * mu
    inv = g_ref[...] / jnp.sqrt(var + EPS)
    cc = be_ref[...] - mu * inv
    x = jnp.maximum(y_ref[...] * inv + cc, 0.0)
    y3 = (
        jnp.dot(x, w_ref[...], preferred_element_type=jnp.float32)
        + b3_ref[...]
    )
    o_ref[...] = y3

    @pl.when(pl.program_id(0) == 0)
    def _():
        s3_ref[...] = jnp.zeros_like(s3_ref)

    s3_ref[0:1, :] += jnp.sum(y3, axis=0)[None, :]
    s3_ref[1:2, :] += jnp.sum(y3 * y3, axis=0)[None, :]


def _run_mm3(y2, s2, g2r, be2r, w3t, b3r):
    nblk = 32
    rows = P // nblk
    return pl.pallas_call(
        _mm3_body,
        grid=(nblk,),
        in_specs=[
            pl.BlockSpec((rows, C), lambda i: (i, 0)),
            pl.BlockSpec((8, C), lambda i: (0, 0)),
            pl.BlockSpec((1, C), lambda i: (0, 0)),
            pl.BlockSpec((1, C), lambda i: (0, 0)),
            pl.BlockSpec((C, C3), lambda i: (0, 0)),
            pl.BlockSpec((1, C3), lambda i: (0, 0)),
        ],
        out_specs=[
            pl.BlockSpec((rows, C3), lambda i: (i, 0)),
            pl.BlockSpec((8, C3), lambda i: (0, 0)),
        ],
        out_shape=[
            jax.ShapeDtypeStruct((P, C3), jnp.float32),
            jax.ShapeDtypeStruct((8, C3), jnp.float32),
        ],
    )(y2, s2, g2r, be2r, w3t, b3r)


def _final_body(y_ref, s_ref, g_ref, be_ref, o_ref):
    mu = s_ref[0:1, :] / BN_N
    var = s_ref[1:2, :] / BN_N - mu * mu
    inv = g_ref[...] / jnp.sqrt(var + EPS)
    cc = be_ref[...] - mu * inv
    z = jnp.maximum(y_ref[...] * inv.reshape(1, 1, C3) + cc.reshape(1, 1, C3), 0.0)
    o_ref[...] = jnp.max(z, axis=1)


def _run_final(y3_3d, s3, g3r, be3r):
    nblk = 32
    g = (B * M) // nblk
    return pl.pallas_call(
        _final_body,
        grid=(nblk,),
        in_specs=[
            pl.BlockSpec((g, K, C3), lambda i: (i, 0, 0)),
            pl.BlockSpec((8, C3), lambda i: (0, 0)),
            pl.BlockSpec((1, C3), lambda i: (0, 0)),
            pl.BlockSpec((1, C3), lambda i: (0, 0)),
        ],
        out_specs=pl.BlockSpec((g, C3), lambda i: (i, 0)),
        out_shape=jax.ShapeDtypeStruct((B * M, C3), jnp.float32),
    )(y3_3d, s3, g3r, be3r)


def kernel(xyz, features, W1, b1, g1, be1, W2, b2, g2, be2, W3, b3, g3, be3):
    xt = jnp.transpose(xyz, (2, 0, 1))
    nxt = _run_fps(xt)
    new_xyz = jnp.transpose(nxt, (1, 2, 0))

    w1t = W1.T
    xall = jnp.concatenate([xyz, features], axis=-1).reshape(B * N, 131)
    g2d = _run_mm1(xall, w1t, b1.reshape(1, C))

    idx, v = _run_ball(new_xyz, xt, w1t)

    y1 = _gather_rows(g2d, idx.reshape(P // 128, 128))
    y1_3d = y1.reshape(B * M, K, C)
    v2d = v.reshape(B * M, C)

    s1 = _run_stats1(y1_3d, v2d)
    y2, s2 = _run_mm2(
        y1_3d, v2d, s1, g1.reshape(1, C), be1.reshape(1, C), W2.T,
        b2.reshape(1, C),
    )
    y3, s3 = _run_mm3(
        y2, s2, g2.reshape(1, C), be2.reshape(1, C), W3.T, b3.reshape(1, C3)
    )
    nf = _run_final(
        y3.reshape(B * M, K, C3), s3, g3.reshape(1, C3), be3.reshape(1, C3)
    )
    return new_xyz, nf.reshape(B, M, C3)


# speedup vs baseline: 12.3938x; 1.0027x over previous
import functools

import jax
import jax.numpy as jnp
from jax import lax
from jax.experimental import pallas as pl
from jax.experimental.pallas import tpu as pltpu
from jax.experimental.pallas import tpu_sc as plsc

B, N, M, K, C = 8, 4096, 512, 32, 128
R2 = 0.2 ** 2
P = B * M * K
BN_N = float(P)
EPS = 1e-5
C3 = 256


def _fps_body(xt_ref, nxt_ref):
    x = xt_ref[0]
    y = xt_ref[1]
    z = xt_ref[2]
    lanes_n = lax.broadcasted_iota(jnp.int32, (B, N), 1)
    lanes_m = lax.broadcasted_iota(jnp.int32, (B, M), 1)

    def body(i, carry):
        dist, far, ax, ay, az = carry
        sel = lanes_n == far
        cx = jnp.sum(jnp.where(sel, x, 0.0), axis=1, keepdims=True)
        cy = jnp.sum(jnp.where(sel, y, 0.0), axis=1, keepdims=True)
        cz = jnp.sum(jnp.where(sel, z, 0.0), axis=1, keepdims=True)
        put = lanes_m == i
        ax = jnp.where(put, cx, ax)
        ay = jnp.where(put, cy, ay)
        az = jnp.where(put, cz, az)
        d = (x - cx) ** 2 + (y - cy) ** 2 + (z - cz) ** 2
        dist = jnp.minimum(dist, d)
        mx = jnp.max(dist, axis=1, keepdims=True)
        far = jnp.min(jnp.where(dist == mx, lanes_n, N), axis=1, keepdims=True)
        return dist, far, ax, ay, az

    dist0 = jnp.full((B, N), 1e10, jnp.float32)
    far0 = jnp.zeros((B, 1), jnp.int32)
    a0 = jnp.zeros((B, M), jnp.float32)
    _, _, ax, ay, az = lax.fori_loop(
        0, M, body, (dist0, far0, a0, a0, a0), unroll=4)
    nxt_ref[0] = ax
    nxt_ref[1] = ay
    nxt_ref[2] = az


def _run_fps(xt):
    return pl.pallas_call(
        _fps_body,
        out_shape=jax.ShapeDtypeStruct((3, B, M), jnp.float32),
    )(xt)


def _ball_body(nxyz_ref, xt_ref, w1t_ref, idx_ref, v_ref, dm_ref):
    b = pl.program_id(0)
    nx = nxyz_ref[0]
    v = jnp.zeros((M, C), jnp.float32)
    for j in range(3):
        v = v + nx[:, j:j + 1] * w1t_ref[j:j + 1, :]
    v_ref[0] = v
    def bf(t):
        return t.astype(jnp.bfloat16).astype(jnp.float32)

    n0, n1, n2 = nx[:, 0:1], nx[:, 1:2], nx[:, 2:3]
    p0, p1, p2 = xt_ref[0, 0], xt_ref[1, 0], xt_ref[2, 0]
    cross = (bf(n0) * bf(p0) + bf(n1) * bf(p1)) + bf(n2) * bf(p2)
    d = -2.0 * cross
    d = d + ((n0 * n0 + n1 * n1) + n2 * n2)
    d = d + ((p0 * p0 + p1 * p1) + p2 * p2)
    d = jnp.maximum(d, 0.0)
    dm_ref[...] = jnp.where(d > R2, jnp.inf, d)
    lanes_n = lax.broadcasted_iota(jnp.int32, (M, N), 1)
    lanes_k = lax.broadcasted_iota(jnp.int32, (M, K), 1)
    base = b * N

    NSUB = 4

    def body(k, carry):
        acc, idx0 = carry
        dmv = dm_ref[...]
        for t in range(NSUB):
            mn = jnp.min(dmv, axis=1, keepdims=True)
            sel = jnp.min(
                jnp.where(dmv == mn, lanes_n, N), axis=1, keepdims=True)
            idx0 = jnp.where((k == 0) & (t == 0), sel, idx0)
            chosen = jnp.where(mn == jnp.inf, idx0, sel) + base
            acc = jnp.where(lanes_k == k * NSUB + t, chosen, acc)
            dmv = jnp.where(lanes_n == sel, jnp.inf, dmv)
        dm_ref[...] = dmv
        return acc, idx0

    acc0 = jnp.zeros((M, K), jnp.int32)
    acc, _ = lax.fori_loop(
        0, K // NSUB, body, (acc0, jnp.zeros((M, 1), jnp.int32)))
    idx_ref[0] = acc


def _run_ball(new_xyz, xt, w1t):
    xt4 = xt.reshape(3, B, 1, N)
    return pl.pallas_call(
        _ball_body,
        grid=(B,),
        in_specs=[
            pl.BlockSpec((1, M, 3), lambda i: (i, 0, 0)),
            pl.BlockSpec((3, 1, 1, N), lambda i: (0, i, 0, 0)),
            pl.BlockSpec((131, C), lambda i: (0, 0)),
        ],
        out_specs=[
            pl.BlockSpec((1, M, K), lambda i: (i, 0, 0)),
            pl.BlockSpec((1, M, C), lambda i: (i, 0, 0)),
        ],
        out_shape=[
            jax.ShapeDtypeStruct((B, M, K), jnp.int32),
            jax.ShapeDtypeStruct((B, M, C), jnp.float32),
        ],
        scratch_shapes=[pltpu.VMEM((M, N), jnp.float32)],
    )(new_xyz, xt4, w1t)


def _mm1_body(x_ref, w_ref, b_ref, o_ref):
    o_ref[...] = (
        jnp.dot(x_ref[...], w_ref[...], preferred_element_type=jnp.float32)
        + b_ref[...]
    )


def _run_mm1(xall, w1t, b1r):
    nblk = 8
    rows = (B * N) // nblk
    return pl.pallas_call(
        _mm1_body,
        grid=(nblk,),
        in_specs=[
            pl.BlockSpec((rows, 131), lambda i: (i, 0)),
            pl.BlockSpec((131, C), lambda i: (0, 0)),
            pl.BlockSpec((1, C), lambda i: (0, 0)),
        ],
        out_specs=pl.BlockSpec((rows, C), lambda i: (i, 0)),
        out_shape=jax.ShapeDtypeStruct((B * N, C), jnp.float32),
    )(xall, w1t, b1r)


_SC_NW = 32
_SC_CHUNK = 128
_SC_NCHUNK = P // _SC_NW // _SC_CHUNK


def _gather_rows(g2d, idx2d):
    mesh = plsc.VectorSubcoreMesh(core_axis_name="c", subcore_axis_name="s")

    @functools.partial(
        pl.kernel,
        mesh=mesh,
        out_type=jax.ShapeDtypeStruct((P, C), jnp.float32),
        scratch_types=[
            pltpu.VMEM((_SC_NCHUNK, _SC_CHUNK), jnp.int32),
            pltpu.VMEM((_SC_CHUNK, C), jnp.float32),
            pltpu.SemaphoreType.DMA,
        ],
    )
    def k(g_hbm, idx_hbm, out_hbm, idx_v, rows_v, sem):
        wid = lax.axis_index("s") * 2 + lax.axis_index("c")
        chunk0 = wid * _SC_NCHUNK
        pltpu.sync_copy(idx_hbm.at[pl.ds(chunk0, _SC_NCHUNK)], idx_v)

        def body(j, _):
            pltpu.async_copy(g_hbm.at[idx_v.at[j]], rows_v, sem).wait()
            pltpu.sync_copy(
                rows_v, out_hbm.at[pl.ds((chunk0 + j) * _SC_CHUNK, _SC_CHUNK)]
            )
            return 0

        lax.fori_loop(0, _SC_NCHUNK, body, 0)

    return k(g2d, idx2d)


def _stats1_body(y_ref, v_ref, s_ref):
    y = y_ref[...] - v_ref[...][:, None, :]

    @pl.when(pl.program_id(0) == 0)
    def _():
        s_ref[...] = jnp.zeros_like(s_ref)

    s_ref[0:1, :] += jnp.sum(y, axis=(0, 1))[None, :]
    s_ref[1:2, :] += jnp.sum(y * y, axis=(0, 1))[None, :]


def _run_stats1(y1_3d, v2d):
    nblk = 32
    g = (B * M) // nblk
    return pl.pallas_call(
        _stats1_body,
        grid=(nblk,),
        in_specs=[
            pl.BlockSpec((g, K, C), lambda i: (i, 0, 0)),
            pl.BlockSpec((g, C), lambda i: (i, 0)),
        ],
        out_specs=pl.BlockSpec((8, C), lambda i: (0, 0)),
        out_shape=jax.ShapeDtypeStruct((8, C), jnp.float32),
    )(y1_3d, v2d)


def _bn_coeffs(s_ref, g_ref, be_ref, cols):
    mu = s_ref[0:1, :] / BN_N
    var = s_ref[1:2, :] / BN_N - mu * mu
    inv = g_ref[...] / jnp.sqrt(var + EPS)
    cc = be_ref[...] - mu * inv
    return inv.reshape(1, 1, cols), cc.reshape(1, 1, cols)


def _mm2_body(y_ref, v_ref, s_ref, g_ref, be_ref, w_ref, b2_ref, o_ref, s2_ref):
    inv, cc = _bn_coeffs(s_ref, g_ref, be_ref, C)
    y = y_ref[...] - v_ref[...][:, None, :]
    x = jnp.maximum(y * inv + cc, 0.0)
    xf = x.reshape(y.shape[0] * K, C)
    y2 = (
        jnp.dot(xf, w_ref[...], preferred_element_type=jnp.float32)
        + b2_ref[...]
    )
    o_ref[...] = y2

    @pl.when(pl.program_id(0) == 0)
    def _():
        s2_ref[...] = jnp.zeros_like(s2_ref)

    s2_ref[0:1, :] += jnp.sum(y2, axis=0)[None, :]
    s2_ref[1:2, :] += jnp.sum(y2 * y2, axis=0)[None, :]


def _run_mm2(y1_3d, v2d, s1, g1r, be1r, w2t, b2r):
    nblk = 32
    g = (B * M) // nblk
    return pl.pallas_call(
        _mm2_body,
        grid=(nblk,),
        in_specs=[
            pl.BlockSpec((g, K, C), lambda i: (i, 0, 0)),
            pl.BlockSpec((g, C), lambda i: (i, 0)),
            pl.BlockSpec((8, C), lambda i: (0, 0)),
            pl.BlockSpec((1, C), lambda i: (0, 0)),
            pl.BlockSpec((1, C), lambda i: (0, 0)),
            pl.BlockSpec((C, C), lambda i: (0, 0)),
            pl.BlockSpec((1, C), lambda i: (0, 0)),
        ],
        out_specs=[
            pl.BlockSpec((g * K, C), lambda i: (i, 0)),
            pl.BlockSpec((8, C), lambda i: (0, 0)),
        ],
        out_shape=[
            jax.ShapeDtypeStruct((P, C), jnp.float32),
            jax.ShapeDtypeStruct((8, C), jnp.float32),
        ],
    )(y1_3d, v2d, s1, g1r, be1r, w2t, b2r)


def _mm3_body(y_ref, s_ref, g_ref, be_ref, w_ref, b3_ref, o_ref, s3_ref):
    mu = s_ref[0:1, :] / BN_N
    var = s_ref[1:2, :] / BN_N - mu * mu
    inv = g_ref[...] / jnp.sqrt(var + EPS)
    cc = be_ref[...] - mu * inv
    x = jnp.maximum(y_ref[...] * inv + cc, 0.0)
    y3 = (
        jnp.dot(x, w_ref[...], preferred_element_type=jnp.float32)
        + b3_ref[...]
    )
    o_ref[...] = y3

    @pl.when(pl.program_id(0) == 0)
    def _():
        s3_ref[...] = jnp.zeros_like(s3_ref)

    s3_ref[0:1, :] += jnp.sum(y3, axis=0)[None, :]
    s3_ref[1:2, :] += jnp.sum(y3 * y3, axis=0)[None, :]


def _run_mm3(y2, s2, g2r, be2r, w3t, b3r):
    nblk = 32
    rows = P // nblk
    return pl.pallas_call(
        _mm3_body,
        grid=(nblk,),
        in_specs=[
            pl.BlockSpec((rows, C), lambda i: (i, 0)),
            pl.BlockSpec((8, C), lambda i: (0, 0)),
            pl.BlockSpec((1, C), lambda i: (0, 0)),
            pl.BlockSpec((1, C), lambda i: (0, 0)),
            pl.BlockSpec((C, C3), lambda i: (0, 0)),
            pl.BlockSpec((1, C3), lambda i: (0, 0)),
        ],
        out_specs=[
            pl.BlockSpec((rows, C3), lambda i: (i, 0)),
            pl.BlockSpec((8, C3), lambda i: (0, 0)),
        ],
        out_shape=[
            jax.ShapeDtypeStruct((P, C3), jnp.float32),
            jax.ShapeDtypeStruct((8, C3), jnp.float32),
        ],
    )(y2, s2, g2r, be2r, w3t, b3r)


def _final_body(y_ref, s_ref, g_ref, be_ref, o_ref):
    mu = s_ref[0:1, :] / BN_N
    var = s_ref[1:2, :] / BN_N - mu * mu
    inv = g_ref[...] / jnp.sqrt(var + EPS)
    cc = be_ref[...] - mu * inv
    z = jnp.maximum(y_ref[...] * inv.reshape(1, 1, C3) + cc.reshape(1, 1, C3), 0.0)
    o_ref[...] = jnp.max(z, axis=1)


def _run_final(y3_3d, s3, g3r, be3r):
    nblk = 32
    g = (B * M) // nblk
    return pl.pallas_call(
        _final_body,
        grid=(nblk,),
        in_specs=[
            pl.BlockSpec((g, K, C3), lambda i: (i, 0, 0)),
            pl.BlockSpec((8, C3), lambda i: (0, 0)),
            pl.BlockSpec((1, C3), lambda i: (0, 0)),
            pl.BlockSpec((1, C3), lambda i: (0, 0)),
        ],
        out_specs=pl.BlockSpec((g, C3), lambda i: (i, 0)),
        out_shape=jax.ShapeDtypeStruct((B * M, C3), jnp.float32),
    )(y3_3d, s3, g3r, be3r)


def kernel(xyz, features, W1, b1, g1, be1, W2, b2, g2, be2, W3, b3, g3, be3):
    xt = jnp.transpose(xyz, (2, 0, 1))
    nxt = _run_fps(xt)
    new_xyz = jnp.transpose(nxt, (1, 2, 0))

    w1t = W1.T
    xall = jnp.concatenate([xyz, features], axis=-1).reshape(B * N, 131)
    g2d = _run_mm1(xall, w1t, b1.reshape(1, C))

    idx, v = _run_ball(new_xyz, xt, w1t)

    y1 = _gather_rows(g2d, idx.reshape(P // 128, 128))
    y1_3d = y1.reshape(B * M, K, C)
    v2d = v.reshape(B * M, C)

    s1 = _run_stats1(y1_3d, v2d)
    y2, s2 = _run_mm2(
        y1_3d, v2d, s1, g1.reshape(1, C), be1.reshape(1, C), W2.T,
        b2.reshape(1, C),
    )
    y3, s3 = _run_mm3(
        y2, s2, g2.reshape(1, C), be2.reshape(1, C), W3.T, b3.reshape(1, C3)
    )
    nf = _run_final(
        y3.reshape(B * M, K, C3), s3, g3.reshape(1, C3), be3.reshape(1, C3)
    )
    return new_xyz, nf.reshape(B, M, C3)


# speedup vs baseline: 12.8164x; 1.0341x over previous
import functools

import jax
import jax.numpy as jnp
from jax import lax
from jax.experimental import pallas as pl
from jax.experimental.pallas import tpu as pltpu
from jax.experimental.pallas import tpu_sc as plsc

B, N, M, K, C = 8, 4096, 512, 32, 128
R2 = 0.2 ** 2
P = B * M * K
BN_N = float(P)
EPS = 1e-5
C3 = 256


def _fps_body(xt_ref, nxt_ref):
    x = xt_ref[0]
    y = xt_ref[1]
    z = xt_ref[2]
    lanes_n = lax.broadcasted_iota(jnp.int32, (B, N), 1)
    lanes_m = lax.broadcasted_iota(jnp.int32, (B, M), 1)

    def body(i, carry):
        dist, far, ax, ay, az = carry
        sel = lanes_n == far
        cx = jnp.sum(jnp.where(sel, x, 0.0), axis=1, keepdims=True)
        cy = jnp.sum(jnp.where(sel, y, 0.0), axis=1, keepdims=True)
        cz = jnp.sum(jnp.where(sel, z, 0.0), axis=1, keepdims=True)
        put = lanes_m == i
        ax = jnp.where(put, cx, ax)
        ay = jnp.where(put, cy, ay)
        az = jnp.where(put, cz, az)
        d = (x - cx) ** 2 + (y - cy) ** 2 + (z - cz) ** 2
        dist = jnp.minimum(dist, d)
        mx = jnp.max(dist, axis=1, keepdims=True)
        far = jnp.min(jnp.where(dist == mx, lanes_n, N), axis=1, keepdims=True)
        return dist, far, ax, ay, az

    dist0 = jnp.full((B, N), 1e10, jnp.float32)
    far0 = jnp.zeros((B, 1), jnp.int32)
    a0 = jnp.zeros((B, M), jnp.float32)
    _, _, ax, ay, az = lax.fori_loop(
        0, M, body, (dist0, far0, a0, a0, a0), unroll=4)
    nxt_ref[0] = ax
    nxt_ref[1] = ay
    nxt_ref[2] = az


def _run_fps(xt):
    return pl.pallas_call(
        _fps_body,
        out_shape=jax.ShapeDtypeStruct((3, B, M), jnp.float32),
    )(xt)


def _ball_body(nxyz_ref, xt_ref, w1t_ref, idx_ref, v_ref, dm_ref):
    b = pl.program_id(0)
    nx = nxyz_ref[0]
    v = jnp.zeros((M, C), jnp.float32)
    for j in range(3):
        v = v + nx[:, j:j + 1] * w1t_ref[j:j + 1, :]
    v_ref[0] = v
    def bf(t):
        return t.astype(jnp.bfloat16).astype(jnp.float32)

    n0, n1, n2 = nx[:, 0:1], nx[:, 1:2], nx[:, 2:3]
    p0, p1, p2 = xt_ref[0, 0], xt_ref[1, 0], xt_ref[2, 0]
    cross = (bf(n0) * bf(p0) + bf(n1) * bf(p1)) + bf(n2) * bf(p2)
    d = -2.0 * cross
    d = d + ((n0 * n0 + n1 * n1) + n2 * n2)
    d = d + ((p0 * p0 + p1 * p1) + p2 * p2)
    d = jnp.maximum(d, 0.0)
    dm_ref[...] = jnp.where(d > R2, jnp.inf, d)
    lanes_n = lax.broadcasted_iota(jnp.int32, (M, N), 1)
    lanes_k = lax.broadcasted_iota(jnp.int32, (M, K), 1)
    base = b * N

    NSUB = 4

    def body(k, carry):
        acc, idx0 = carry
        dmv = dm_ref[...]
        for t in range(NSUB):
            mn = jnp.min(dmv, axis=1, keepdims=True)
            sel = jnp.min(
                jnp.where(dmv == mn, lanes_n, N), axis=1, keepdims=True)
            idx0 = jnp.where((k == 0) & (t == 0), sel, idx0)
            chosen = jnp.where(mn == jnp.inf, idx0, sel) + base
            acc = jnp.where(lanes_k == k * NSUB + t, chosen, acc)
            dmv = jnp.where(lanes_n == sel, jnp.inf, dmv)
        dm_ref[...] = dmv
        return acc, idx0

    acc0 = jnp.zeros((M, K), jnp.int32)
    acc, _ = lax.fori_loop(
        0, K // NSUB, body, (acc0, jnp.zeros((M, 1), jnp.int32)))
    idx_ref[0] = acc


def _run_ball(new_xyz, xt, w1t):
    xt4 = xt.reshape(3, B, 1, N)
    return pl.pallas_call(
        _ball_body,
        grid=(B,),
        in_specs=[
            pl.BlockSpec((1, M, 3), lambda i: (i, 0, 0)),
            pl.BlockSpec((3, 1, 1, N), lambda i: (0, i, 0, 0)),
            pl.BlockSpec((131, C), lambda i: (0, 0)),
        ],
        out_specs=[
            pl.BlockSpec((1, M, K), lambda i: (i, 0, 0)),
            pl.BlockSpec((1, M, C), lambda i: (i, 0, 0)),
        ],
        out_shape=[
            jax.ShapeDtypeStruct((B, M, K), jnp.int32),
            jax.ShapeDtypeStruct((B, M, C), jnp.float32),
        ],
        scratch_shapes=[pltpu.VMEM((M, N), jnp.float32)],
    )(new_xyz, xt4, w1t)


def _mm1_body(x_ref, w_ref, b_ref, o_ref):
    o_ref[...] = (
        jnp.dot(x_ref[...], w_ref[...], preferred_element_type=jnp.float32)
        + b_ref[...]
    )


def _run_mm1(xall, w1t, b1r):
    nblk = 8
    rows = (B * N) // nblk
    return pl.pallas_call(
        _mm1_body,
        grid=(nblk,),
        in_specs=[
            pl.BlockSpec((rows, 131), lambda i: (i, 0)),
            pl.BlockSpec((131, C), lambda i: (0, 0)),
            pl.BlockSpec((1, C), lambda i: (0, 0)),
        ],
        out_specs=pl.BlockSpec((rows, C), lambda i: (i, 0)),
        out_shape=jax.ShapeDtypeStruct((B * N, C), jnp.float32),
    )(xall, w1t, b1r)


_SC_NW = 32
_SC_CHUNK = 128
_SC_NCHUNK = P // _SC_NW // _SC_CHUNK


def _gather_rows(g2d, idx2d):
    mesh = plsc.VectorSubcoreMesh(core_axis_name="c", subcore_axis_name="s")

    @functools.partial(
        pl.kernel,
        mesh=mesh,
        out_type=jax.ShapeDtypeStruct((P, C), jnp.float32),
        scratch_types=[
            pltpu.VMEM((_SC_NCHUNK, _SC_CHUNK), jnp.int32),
            pltpu.VMEM((_SC_CHUNK, C), jnp.float32),
            pltpu.SemaphoreType.DMA,
        ],
    )
    def k(g_hbm, idx_hbm, out_hbm, idx_v, rows_v, sem):
        wid = lax.axis_index("s") * 2 + lax.axis_index("c")
        chunk0 = wid * _SC_NCHUNK
        pltpu.sync_copy(idx_hbm.at[pl.ds(chunk0, _SC_NCHUNK)], idx_v)

        def body(j, _):
            pltpu.async_copy(g_hbm.at[idx_v.at[j]], rows_v, sem).wait()
            pltpu.sync_copy(
                rows_v, out_hbm.at[pl.ds((chunk0 + j) * _SC_CHUNK, _SC_CHUNK)]
            )
            return 0

        lax.fori_loop(0, _SC_NCHUNK, body, 0)

    return k(g2d, idx2d)


def _stats1_body(y_ref, v_ref, s_ref):
    y = y_ref[...] - v_ref[...][:, None, :]

    @pl.when(pl.program_id(0) == 0)
    def _():
        s_ref[...] = jnp.zeros_like(s_ref)

    s_ref[0:1, :] += jnp.sum(y, axis=(0, 1))[None, :]
    s_ref[1:2, :] += jnp.sum(y * y, axis=(0, 1))[None, :]


def _run_stats1(y1_3d, v2d):
    nblk = 32
    g = (B * M) // nblk
    return pl.pallas_call(
        _stats1_body,
        grid=(nblk,),
        in_specs=[
            pl.BlockSpec((g, K, C), lambda i: (i, 0, 0)),
            pl.BlockSpec((g, C), lambda i: (i, 0)),
        ],
        out_specs=pl.BlockSpec((8, C), lambda i: (0, 0)),
        out_shape=jax.ShapeDtypeStruct((8, C), jnp.float32),
    )(y1_3d, v2d)


def _bn_coeffs(s_ref, g_ref, be_ref, cols):
    mu = s_ref[0:1, :] / BN_N
    var = s_ref[1:2, :] / BN_N - mu * mu
    inv = g_ref[...] / jnp.sqrt(var + EPS)
    cc = be_ref[...] - mu * inv
    return inv.reshape(1, 1, cols), cc.reshape(1, 1, cols)


def _mm2_body(y_ref, v_ref, s_ref, g_ref, be_ref, w_ref, b2_ref, o_ref, s2_ref):
    inv, cc = _bn_coeffs(s_ref, g_ref, be_ref, C)
    y = y_ref[...] - v_ref[...][:, None, :]
    x = jnp.maximum(y * inv + cc, 0.0)
    xf = x.reshape(y.shape[0] * K, C)
    y2 = (
        jnp.dot(xf, w_ref[...], preferred_element_type=jnp.float32)
        + b2_ref[...]
    )
    o_ref[...] = y2.astype(jnp.bfloat16)

    @pl.when(pl.program_id(0) == 0)
    def _():
        s2_ref[...] = jnp.zeros_like(s2_ref)

    s2_ref[0:1, :] += jnp.sum(y2, axis=0)[None, :]
    s2_ref[1:2, :] += jnp.sum(y2 * y2, axis=0)[None, :]


def _run_mm2(y1_3d, v2d, s1, g1r, be1r, w2t, b2r):
    nblk = 32
    g = (B * M) // nblk
    return pl.pallas_call(
        _mm2_body,
        grid=(nblk,),
        in_specs=[
            pl.BlockSpec((g, K, C), lambda i: (i, 0, 0)),
            pl.BlockSpec((g, C), lambda i: (i, 0)),
            pl.BlockSpec((8, C), lambda i: (0, 0)),
            pl.BlockSpec((1, C), lambda i: (0, 0)),
            pl.BlockSpec((1, C), lambda i: (0, 0)),
            pl.BlockSpec((C, C), lambda i: (0, 0)),
            pl.BlockSpec((1, C), lambda i: (0, 0)),
        ],
        out_specs=[
            pl.BlockSpec((g * K, C), lambda i: (i, 0)),
            pl.BlockSpec((8, C), lambda i: (0, 0)),
        ],
        out_shape=[
            jax.ShapeDtypeStruct((P, C), jnp.bfloat16),
            jax.ShapeDtypeStruct((8, C), jnp.float32),
        ],
    )(y1_3d, v2d, s1, g1r, be1r, w2t, b2r)


def _mm3_body(y_ref, s_ref, g_ref, be_ref, w_ref, b3_ref, o_ref, s3_ref):
    mu = s_ref[0:1, :] / BN_N
    var = s_ref[1:2, :] / BN_N - mu * mu
    inv = g_ref[...] / jnp.sqrt(var + EPS)
    cc = be_ref[...] - mu * inv
    x = jnp.maximum(y_ref[...].astype(jnp.float32) * inv + cc, 0.0)
    y3 = (
        jnp.dot(x, w_ref[...], preferred_element_type=jnp.float32)
        + b3_ref[...]
    )
    o_ref[...] = y3.astype(jnp.bfloat16)

    @pl.when(pl.program_id(0) == 0)
    def _():
        s3_ref[...] = jnp.zeros_like(s3_ref)

    s3_ref[0:1, :] += jnp.sum(y3, axis=0)[None, :]
    s3_ref[1:2, :] += jnp.sum(y3 * y3, axis=0)[None, :]


def _run_mm3(y2, s2, g2r, be2r, w3t, b3r):
    nblk = 32
    rows = P // nblk
    return pl.pallas_call(
        _mm3_body,
        grid=(nblk,),
        in_specs=[
            pl.BlockSpec((rows, C), lambda i: (i, 0)),
            pl.BlockSpec((8, C), lambda i: (0, 0)),
            pl.BlockSpec((1, C), lambda i: (0, 0)),
            pl.BlockSpec((1, C), lambda i: (0, 0)),
            pl.BlockSpec((C, C3), lambda i: (0, 0)),
            pl.BlockSpec((1, C3), lambda i: (0, 0)),
        ],
        out_specs=[
            pl.BlockSpec((rows, C3), lambda i: (i, 0)),
            pl.BlockSpec((8, C3), lambda i: (0, 0)),
        ],
        out_shape=[
            jax.ShapeDtypeStruct((P, C3), jnp.bfloat16),
            jax.ShapeDtypeStruct((8, C3), jnp.float32),
        ],
    )(y2, s2, g2r, be2r, w3t, b3r)


def _final_body(y_ref, s_ref, g_ref, be_ref, o_ref):
    mu = s_ref[0:1, :] / BN_N
    var = s_ref[1:2, :] / BN_N - mu * mu
    inv = g_ref[...] / jnp.sqrt(var + EPS)
    cc = be_ref[...] - mu * inv
    z = jnp.maximum(y_ref[...].astype(jnp.float32) * inv.reshape(1, 1, C3)
                    + cc.reshape(1, 1, C3), 0.0)
    o_ref[...] = jnp.max(z, axis=1)


def _run_final(y3_3d, s3, g3r, be3r):
    nblk = 32
    g = (B * M) // nblk
    return pl.pallas_call(
        _final_body,
        grid=(nblk,),
        in_specs=[
            pl.BlockSpec((g, K, C3), lambda i: (i, 0, 0)),
            pl.BlockSpec((8, C3), lambda i: (0, 0)),
            pl.BlockSpec((1, C3), lambda i: (0, 0)),
            pl.BlockSpec((1, C3), lambda i: (0, 0)),
        ],
        out_specs=pl.BlockSpec((g, C3), lambda i: (i, 0)),
        out_shape=jax.ShapeDtypeStruct((B * M, C3), jnp.float32),
    )(y3_3d, s3, g3r, be3r)


def kernel(xyz, features, W1, b1, g1, be1, W2, b2, g2, be2, W3, b3, g3, be3):
    xt = jnp.transpose(xyz, (2, 0, 1))
    nxt = _run_fps(xt)
    new_xyz = jnp.transpose(nxt, (1, 2, 0))

    w1t = W1.T
    xall = jnp.concatenate([xyz, features], axis=-1).reshape(B * N, 131)
    g2d = _run_mm1(xall, w1t, b1.reshape(1, C))

    idx, v = _run_ball(new_xyz, xt, w1t)

    y1 = _gather_rows(g2d, idx.reshape(P // 128, 128))
    y1_3d = y1.reshape(B * M, K, C)
    v2d = v.reshape(B * M, C)

    s1 = _run_stats1(y1_3d, v2d)
    y2, s2 = _run_mm2(
        y1_3d, v2d, s1, g1.reshape(1, C), be1.reshape(1, C), W2.T,
        b2.reshape(1, C),
    )
    y3, s3 = _run_mm3(
        y2, s2, g2.reshape(1, C), be2.reshape(1, C), W3.T, b3.reshape(1, C3)
    )
    nf = _run_final(
        y3.reshape(B * M, K, C3), s3, g3.reshape(1, C3), be3.reshape(1, C3)
    )
    return new_xyz, nf.reshape(B, M, C3)


# speedup vs baseline: 12.9282x; 1.0087x over previous
import functools

import jax
import jax.numpy as jnp
from jax import lax
from jax.experimental import pallas as pl
from jax.experimental.pallas import tpu as pltpu
from jax.experimental.pallas import tpu_sc as plsc

B, N, M, K, C = 8, 4096, 512, 32, 128
R2 = 0.2 ** 2
P = B * M * K
BN_N = float(P)
EPS = 1e-5
C3 = 256


def _fps_body(xt_ref, nxt_ref):
    x = xt_ref[0]
    y = xt_ref[1]
    z = xt_ref[2]
    lanes_n = lax.broadcasted_iota(jnp.int32, (B, N), 1)
    lanes_m = lax.broadcasted_iota(jnp.int32, (B, M), 1)

    def body(i, carry):
        dist, far, ax, ay, az = carry
        sel = lanes_n == far
        cx = jnp.sum(jnp.where(sel, x, 0.0), axis=1, keepdims=True)
        cy = jnp.sum(jnp.where(sel, y, 0.0), axis=1, keepdims=True)
        cz = jnp.sum(jnp.where(sel, z, 0.0), axis=1, keepdims=True)
        put = lanes_m == i
        ax = jnp.where(put, cx, ax)
        ay = jnp.where(put, cy, ay)
        az = jnp.where(put, cz, az)
        d = (x - cx) ** 2 + (y - cy) ** 2 + (z - cz) ** 2
        dist = jnp.minimum(dist, d)
        mx = jnp.max(dist, axis=1, keepdims=True)
        far = jnp.min(jnp.where(dist == mx, lanes_n, N), axis=1, keepdims=True)
        return dist, far, ax, ay, az

    dist0 = jnp.full((B, N), 1e10, jnp.float32)
    far0 = jnp.zeros((B, 1), jnp.int32)
    a0 = jnp.zeros((B, M), jnp.float32)
    _, _, ax, ay, az = lax.fori_loop(
        0, M, body, (dist0, far0, a0, a0, a0), unroll=4)
    nxt_ref[0] = ax
    nxt_ref[1] = ay
    nxt_ref[2] = az


def _run_fps(xt):
    return pl.pallas_call(
        _fps_body,
        out_shape=jax.ShapeDtypeStruct((3, B, M), jnp.float32),
    )(xt)


def _ball_body(nxyz_ref, xt_ref, w1t_ref, idx_ref, v_ref, dm_ref):
    b = pl.program_id(0)
    nx = nxyz_ref[0]
    v = jnp.zeros((M, C), jnp.float32)
    for j in range(3):
        v = v + nx[:, j:j + 1] * w1t_ref[j:j + 1, :]
    v_ref[0] = v
    def bf(t):
        return t.astype(jnp.bfloat16).astype(jnp.float32)

    n0, n1, n2 = nx[:, 0:1], nx[:, 1:2], nx[:, 2:3]
    p0, p1, p2 = xt_ref[0, 0], xt_ref[1, 0], xt_ref[2, 0]
    cross = (bf(n0) * bf(p0) + bf(n1) * bf(p1)) + bf(n2) * bf(p2)
    d = -2.0 * cross
    d = d + ((n0 * n0 + n1 * n1) + n2 * n2)
    d = d + ((p0 * p0 + p1 * p1) + p2 * p2)
    d = jnp.maximum(d, 0.0)
    dm_ref[...] = jnp.where(d > R2, jnp.inf, d)
    lanes_n = lax.broadcasted_iota(jnp.int32, (M, N), 1)
    lanes_k = lax.broadcasted_iota(jnp.int32, (M, K), 1)
    base = b * N

    NSUB = 8

    def body(k, carry):
        acc, idx0 = carry
        dmv = dm_ref[...]
        for t in range(NSUB):
            mn = jnp.min(dmv, axis=1, keepdims=True)
            sel = jnp.min(
                jnp.where(dmv == mn, lanes_n, N), axis=1, keepdims=True)
            idx0 = jnp.where((k == 0) & (t == 0), sel, idx0)
            chosen = jnp.where(mn == jnp.inf, idx0, sel) + base
            acc = jnp.where(lanes_k == k * NSUB + t, chosen, acc)
            dmv = jnp.where(lanes_n == sel, jnp.inf, dmv)
        dm_ref[...] = dmv
        return acc, idx0

    acc0 = jnp.zeros((M, K), jnp.int32)
    acc, _ = lax.fori_loop(
        0, K // NSUB, body, (acc0, jnp.zeros((M, 1), jnp.int32)))
    idx_ref[0] = acc


def _run_ball(new_xyz, xt, w1t):
    xt4 = xt.reshape(3, B, 1, N)
    return pl.pallas_call(
        _ball_body,
        grid=(B,),
        in_specs=[
            pl.BlockSpec((1, M, 3), lambda i: (i, 0, 0)),
            pl.BlockSpec((3, 1, 1, N), lambda i: (0, i, 0, 0)),
            pl.BlockSpec((131, C), lambda i: (0, 0)),
        ],
        out_specs=[
            pl.BlockSpec((1, M, K), lambda i: (i, 0, 0)),
            pl.BlockSpec((1, M, C), lambda i: (i, 0, 0)),
        ],
        out_shape=[
            jax.ShapeDtypeStruct((B, M, K), jnp.int32),
            jax.ShapeDtypeStruct((B, M, C), jnp.float32),
        ],
        scratch_shapes=[pltpu.VMEM((M, N), jnp.float32)],
    )(new_xyz, xt4, w1t)


def _mm1_body(x_ref, w_ref, b_ref, o_ref):
    o_ref[...] = (
        jnp.dot(x_ref[...], w_ref[...], preferred_element_type=jnp.float32)
        + b_ref[...]
    )


def _run_mm1(xall, w1t, b1r):
    nblk = 8
    rows = (B * N) // nblk
    return pl.pallas_call(
        _mm1_body,
        grid=(nblk,),
        in_specs=[
            pl.BlockSpec((rows, 131), lambda i: (i, 0)),
            pl.BlockSpec((131, C), lambda i: (0, 0)),
            pl.BlockSpec((1, C), lambda i: (0, 0)),
        ],
        out_specs=pl.BlockSpec((rows, C), lambda i: (i, 0)),
        out_shape=jax.ShapeDtypeStruct((B * N, C), jnp.float32),
    )(xall, w1t, b1r)


_SC_NW = 32
_SC_CHUNK = 128
_SC_NCHUNK = P // _SC_NW // _SC_CHUNK


def _gather_rows(g2d, idx2d):
    mesh = plsc.VectorSubcoreMesh(core_axis_name="c", subcore_axis_name="s")

    @functools.partial(
        pl.kernel,
        mesh=mesh,
        out_type=jax.ShapeDtypeStruct((P, C), jnp.float32),
        scratch_types=[
            pltpu.VMEM((_SC_NCHUNK, _SC_CHUNK), jnp.int32),
            pltpu.VMEM((_SC_CHUNK, C), jnp.float32),
            pltpu.SemaphoreType.DMA,
        ],
    )
    def k(g_hbm, idx_hbm, out_hbm, idx_v, rows_v, sem):
        wid = lax.axis_index("s") * 2 + lax.axis_index("c")
        chunk0 = wid * _SC_NCHUNK
        pltpu.sync_copy(idx_hbm.at[pl.ds(chunk0, _SC_NCHUNK)], idx_v)

        def body(j, _):
            pltpu.async_copy(g_hbm.at[idx_v.at[j]], rows_v, sem).wait()
            pltpu.sync_copy(
                rows_v, out_hbm.at[pl.ds((chunk0 + j) * _SC_CHUNK, _SC_CHUNK)]
            )
            return 0

        lax.fori_loop(0, _SC_NCHUNK, body, 0)

    return k(g2d, idx2d)


def _stats1_body(y_ref, v_ref, s_ref):
    y = y_ref[...] - v_ref[...][:, None, :]

    @pl.when(pl.program_id(0) == 0)
    def _():
        s_ref[...] = jnp.zeros_like(s_ref)

    s_ref[0:1, :] += jnp.sum(y, axis=(0, 1))[None, :]
    s_ref[1:2, :] += jnp.sum(y * y, axis=(0, 1))[None, :]


def _run_stats1(y1_3d, v2d):
    nblk = 32
    g = (B * M) // nblk
    return pl.pallas_call(
        _stats1_body,
        grid=(nblk,),
        in_specs=[
            pl.BlockSpec((g, K, C), lambda i: (i, 0, 0)),
            pl.BlockSpec((g, C), lambda i: (i, 0)),
        ],
        out_specs=pl.BlockSpec((8, C), lambda i: (0, 0)),
        out_shape=jax.ShapeDtypeStruct((8, C), jnp.float32),
    )(y1_3d, v2d)


def _bn_coeffs(s_ref, g_ref, be_ref, cols):
    mu = s_ref[0:1, :] / BN_N
    var = s_ref[1:2, :] / BN_N - mu * mu
    inv = g_ref[...] / jnp.sqrt(var + EPS)
    cc = be_ref[...] - mu * inv
    return inv.reshape(1, 1, cols), cc.reshape(1, 1, cols)


def _mm2_body(y_ref, v_ref, s_ref, g_ref, be_ref, w_ref, b2_ref, o_ref, s2_ref):
    inv, cc = _bn_coeffs(s_ref, g_ref, be_ref, C)
    y = y_ref[...] - v_ref[...][:, None, :]
    x = jnp.maximum(y * inv + cc, 0.0)
    xf = x.reshape(y.shape[0] * K, C)
    y2 = (
        jnp.dot(xf, w_ref[...], preferred_element_type=jnp.float32)
        + b2_ref[...]
    )
    o_ref[...] = y2.astype(jnp.bfloat16)

    @pl.when(pl.program_id(0) == 0)
    def _():
        s2_ref[...] = jnp.zeros_like(s2_ref)

    s2_ref[0:1, :] += jnp.sum(y2, axis=0)[None, :]
    s2_ref[1:2, :] += jnp.sum(y2 * y2, axis=0)[None, :]


def _run_mm2(y1_3d, v2d, s1, g1r, be1r, w2t, b2r):
    nblk = 32
    g = (B * M) // nblk
    return pl.pallas_call(
        _mm2_body,
        grid=(nblk,),
        in_specs=[
            pl.BlockSpec((g, K, C), lambda i: (i, 0, 0)),
            pl.BlockSpec((g, C), lambda i: (i, 0)),
            pl.BlockSpec((8, C), lambda i: (0, 0)),
            pl.BlockSpec((1, C), lambda i: (0, 0)),
            pl.BlockSpec((1, C), lambda i: (0, 0)),
            pl.BlockSpec((C, C), lambda i: (0, 0)),
            pl.BlockSpec((1, C), lambda i: (0, 0)),
        ],
        out_specs=[
            pl.BlockSpec((g * K, C), lambda i: (i, 0)),
            pl.BlockSpec((8, C), lambda i: (0, 0)),
        ],
        out_shape=[
            jax.ShapeDtypeStruct((P, C), jnp.bfloat16),
            jax.ShapeDtypeStruct((8, C), jnp.float32),
        ],
    )(y1_3d, v2d, s1, g1r, be1r, w2t, b2r)


def _mm3_body(y_ref, s_ref, g_ref, be_ref, w_ref, b3_ref, o_ref, s3_ref):
    mu = s_ref[0:1, :] / BN_N
    var = s_ref[1:2, :] / BN_N - mu * mu
    inv = g_ref[...] / jnp.sqrt(var + EPS)
    cc = be_ref[...] - mu * inv
    x = jnp.maximum(y_ref[...].astype(jnp.float32) * inv + cc, 0.0)
    y3 = (
        jnp.dot(x, w_ref[...], preferred_element_type=jnp.float32)
        + b3_ref[...]
    )
    o_ref[...] = y3.astype(jnp.bfloat16)

    @pl.when(pl.program_id(0) == 0)
    def _():
        s3_ref[...] = jnp.zeros_like(s3_ref)

    s3_ref[0:1, :] += jnp.sum(y3, axis=0)[None, :]
    s3_ref[1:2, :] += jnp.sum(y3 * y3, axis=0)[None, :]


def _run_mm3(y2, s2, g2r, be2r, w3t, b3r):
    nblk = 32
    rows = P // nblk
    return pl.pallas_call(
        _mm3_body,
        grid=(nblk,),
        in_specs=[
            pl.BlockSpec((rows, C), lambda i: (i, 0)),
            pl.BlockSpec((8, C), lambda i: (0, 0)),
            pl.BlockSpec((1, C), lambda i: (0, 0)),
            pl.BlockSpec((1, C), lambda i: (0, 0)),
            pl.BlockSpec((C, C3), lambda i: (0, 0)),
            pl.BlockSpec((1, C3), lambda i: (0, 0)),
        ],
        out_specs=[
            pl.BlockSpec((rows, C3), lambda i: (i, 0)),
            pl.BlockSpec((8, C3), lambda i: (0, 0)),
        ],
        out_shape=[
            jax.ShapeDtypeStruct((P, C3), jnp.bfloat16),
            jax.ShapeDtypeStruct((8, C3), jnp.float32),
        ],
    )(y2, s2, g2r, be2r, w3t, b3r)


def _final_body(y_ref, s_ref, g_ref, be_ref, o_ref):
    mu = s_ref[0:1, :] / BN_N
    var = s_ref[1:2, :] / BN_N - mu * mu
    inv = g_ref[...] / jnp.sqrt(var + EPS)
    cc = be_ref[...] - mu * inv
    z = jnp.maximum(y_ref[...].astype(jnp.float32) * inv.reshape(1, 1, C3)
                    + cc.reshape(1, 1, C3), 0.0)
    o_ref[...] = jnp.max(z, axis=1)


def _run_final(y3_3d, s3, g3r, be3r):
    nblk = 32
    g = (B * M) // nblk
    return pl.pallas_call(
        _final_body,
        grid=(nblk,),
        in_specs=[
            pl.BlockSpec((g, K, C3), lambda i: (i, 0, 0)),
            pl.BlockSpec((8, C3), lambda i: (0, 0)),
            pl.BlockSpec((1, C3), lambda i: (0, 0)),
            pl.BlockSpec((1, C3), lambda i: (0, 0)),
        ],
        out_specs=pl.BlockSpec((g, C3), lambda i: (i, 0)),
        out_shape=jax.ShapeDtypeStruct((B * M, C3), jnp.float32),
    )(y3_3d, s3, g3r, be3r)


def kernel(xyz, features, W1, b1, g1, be1, W2, b2, g2, be2, W3, b3, g3, be3):
    xt = jnp.transpose(xyz, (2, 0, 1))
    nxt = _run_fps(xt)
    new_xyz = jnp.transpose(nxt, (1, 2, 0))

    w1t = W1.T
    xall = jnp.concatenate([xyz, features], axis=-1).reshape(B * N, 131)
    g2d = _run_mm1(xall, w1t, b1.reshape(1, C))

    idx, v = _run_ball(new_xyz, xt, w1t)

    y1 = _gather_rows(g2d, idx.reshape(P // 128, 128))
    y1_3d = y1.reshape(B * M, K, C)
    v2d = v.reshape(B * M, C)

    s1 = _run_stats1(y1_3d, v2d)
    y2, s2 = _run_mm2(
        y1_3d, v2d, s1, g1.reshape(1, C), be1.reshape(1, C), W2.T,
        b2.reshape(1, C),
    )
    y3, s3 = _run_mm3(
        y2, s2, g2.reshape(1, C), be2.reshape(1, C), W3.T, b3.reshape(1, C3)
    )
    nf = _run_final(
        y3.reshape(B * M, K, C3), s3, g3.reshape(1, C3), be3.reshape(1, C3)
    )
    return new_xyz, nf.reshape(B, M, C3)
